# baseline (speedup 1.0000x reference)
import jax
import jax.numpy as jnp
import numpy as np
from jax.experimental import pallas as pl
from jax.experimental.pallas import tpu as pltpu

ROWS = 128
COLS = 100000
BLOCK_ROWS = 8
CHUNK = 512
NUM_FULL = COLS // CHUNK
TAIL_BASE = NUM_FULL * CHUNK
TAIL_W = COLS - TAIL_BASE

_TINY = np.float32(np.finfo(np.float32).tiny)
_NEG_INF = np.float32(-np.inf)
_INT_MAX = np.int32(2**31 - 1)


def _threefry_bits(flat_u32):
    u32 = lambda v: jnp.uint32(v)
    ks0 = u32(0)
    ks1 = u32(42)
    ks2 = ks0 ^ ks1 ^ u32(0x1BD11BDA)

    def rotl(x, d):
        return (x << u32(d)) | (x >> u32(32 - d))

    def rounds(x0, x1, rots):
        for r in rots:
            x0 = x0 + x1
            x1 = rotl(x1, r)
            x1 = x0 ^ x1
        return x0, x1

    rot_a = (13, 15, 26, 6)
    rot_b = (17, 29, 16, 24)

    x0 = jnp.zeros_like(flat_u32) + ks0
    x1 = flat_u32 + ks1
    x0, x1 = rounds(x0, x1, rot_a)
    x0 = x0 + ks1
    x1 = x1 + ks2 + u32(1)
    x0, x1 = rounds(x0, x1, rot_b)
    x0 = x0 + ks2
    x1 = x1 + ks0 + u32(2)
    x0, x1 = rounds(x0, x1, rot_a)
    x0 = x0 + ks0
    x1 = x1 + ks1 + u32(3)
    x0, x1 = rounds(x0, x1, rot_b)
    x0 = x0 + ks1
    x1 = x1 + ks2 + u32(4)
    x0, x1 = rounds(x0, x1, rot_a)
    x0 = x0 + ks2
    x1 = x1 + ks0 + u32(5)
    return x0 ^ x1


def _gumbel_from_bits(bits):
    float_bits = (bits >> jnp.uint32(9)) | jnp.uint32(0x3F800000)
    floats = jax.lax.bitcast_convert_type(float_bits, jnp.float32)
    floats = floats - jnp.float32(1.0)
    u = jnp.maximum(_TINY, floats * (jnp.float32(1.0) - _TINY) + _TINY)
    return -jnp.log(-jnp.log(u))


def _sampler_kernel(logits_ref, out_ref):
    i = pl.program_id(0)
    row0 = (i * BLOCK_ROWS).astype(jnp.uint32)
    row_iota = jax.lax.broadcasted_iota(jnp.uint32, (BLOCK_ROWS, 1), 0)
    row_base = (row0 + row_iota) * jnp.uint32(COLS)

    def chunk_vals(base, width):
        col_u = (jnp.uint32(base)
                 + jax.lax.broadcasted_iota(jnp.uint32, (BLOCK_ROWS, width), 1))
        flat = row_base + col_u
        g = _gumbel_from_bits(_threefry_bits(flat))
        return g, col_u

    def merge(carry, vals, col_i32):
        curmax, curidx = carry
        cmax = jnp.max(vals, axis=1, keepdims=True)
        cidx = jnp.min(jnp.where(vals == cmax, col_i32, _INT_MAX),
                       axis=1, keepdims=True)
        take = cmax > curmax
        return (jnp.where(take, cmax, curmax), jnp.where(take, cidx, curidx))

    def body(j, carry):
        base = pl.multiple_of(j * CHUNK, CHUNK)
        g, col_u = chunk_vals(base, CHUNK)
        vals = g + logits_ref[:, pl.ds(base, CHUNK)]
        return merge(carry, vals, col_u.astype(jnp.int32))

    init = (jnp.full((BLOCK_ROWS, 1), _NEG_INF, jnp.float32),
            jnp.zeros((BLOCK_ROWS, 1), jnp.int32))
    curmax, curidx = jax.lax.fori_loop(0, NUM_FULL, body, init)

    g, col_u = chunk_vals(TAIL_BASE, TAIL_W)
    vals = g + logits_ref[:, pl.ds(TAIL_BASE, TAIL_W)]
    curmax, curidx = merge((curmax, curidx), vals, col_u.astype(jnp.int32))

    out_ref[0, 0, :] = curidx[:, 0]


def kernel(logits):
    out = pl.pallas_call(
        _sampler_kernel,
        grid=(ROWS // BLOCK_ROWS,),
        in_specs=[pl.BlockSpec((BLOCK_ROWS, COLS), lambda i: (i, 0))],
        out_specs=pl.BlockSpec((1, 1, BLOCK_ROWS), lambda i: (i, 0, 0)),
        out_shape=jax.ShapeDtypeStruct((ROWS // BLOCK_ROWS, 1, BLOCK_ROWS),
                                       jnp.int32),
        compiler_params=pltpu.CompilerParams(
            dimension_semantics=("arbitrary",)),
    )(logits)
    return out.reshape(ROWS)


# speedup vs baseline: 3.8971x; 3.8971x over previous
import jax
import jax.numpy as jnp
import numpy as np
from jax.experimental import pallas as pl
from jax.experimental.pallas import tpu as pltpu

ROWS = 128
COLS = 100000
BLOCK_ROWS = 8
CHUNK = 1280
NUM_FULL = COLS // CHUNK
TAIL_BASE = NUM_FULL * CHUNK
TAIL_W = COLS - TAIL_BASE

_TINY = np.float32(np.finfo(np.float32).tiny)
_NEG_INF = np.float32(-np.inf)
_INT_MAX = np.int32(2**31 - 1)


def _threefry_bits(flat_u32):
    u32 = lambda v: jnp.uint32(v)
    ks0 = u32(0)
    ks1 = u32(42)
    ks2 = ks0 ^ ks1 ^ u32(0x1BD11BDA)

    def rotl(x, d):
        return (x << u32(d)) | (x >> u32(32 - d))

    def rounds(x0, x1, rots):
        for r in rots:
            x0 = x0 + x1
            x1 = rotl(x1, r)
            x1 = x0 ^ x1
        return x0, x1

    rot_a = (13, 15, 26, 6)
    rot_b = (17, 29, 16, 24)

    x0 = jnp.zeros_like(flat_u32) + ks0
    x1 = flat_u32 + ks1
    x0, x1 = rounds(x0, x1, rot_a)
    x0 = x0 + ks1
    x1 = x1 + ks2 + u32(1)
    x0, x1 = rounds(x0, x1, rot_b)
    x0 = x0 + ks2
    x1 = x1 + ks0 + u32(2)
    x0, x1 = rounds(x0, x1, rot_a)
    x0 = x0 + ks0
    x1 = x1 + ks1 + u32(3)
    x0, x1 = rounds(x0, x1, rot_b)
    x0 = x0 + ks1
    x1 = x1 + ks2 + u32(4)
    x0, x1 = rounds(x0, x1, rot_a)
    x0 = x0 + ks2
    x1 = x1 + ks0 + u32(5)
    return x0 ^ x1


def _gumbel_from_bits(bits):
    float_bits = (bits >> jnp.uint32(9)) | jnp.uint32(0x3F800000)
    floats = jax.lax.bitcast_convert_type(float_bits, jnp.float32)
    floats = floats - jnp.float32(1.0)
    u = jnp.maximum(_TINY, floats * (jnp.float32(1.0) - _TINY) + _TINY)
    return -jnp.log(-jnp.log(u))


def _sampler_kernel(logits_ref, out_ref):
    i = pl.program_id(0)
    row0 = (i * BLOCK_ROWS).astype(jnp.uint32)
    row_iota = jax.lax.broadcasted_iota(jnp.uint32, (BLOCK_ROWS, 1), 0)
    row_base = (row0 + row_iota) * jnp.uint32(COLS)

    def chunk_vals(base, width):
        col_u = (jnp.uint32(base)
                 + jax.lax.broadcasted_iota(jnp.uint32, (BLOCK_ROWS, width), 1))
        flat = row_base + col_u
        g = _gumbel_from_bits(_threefry_bits(flat))
        return g, col_u

    def reduce_first(vecmax, vecidx):
        m = jnp.max(vecmax, axis=1, keepdims=True)
        idx = jnp.min(jnp.where(vecmax == m, vecidx, _INT_MAX),
                      axis=1, keepdims=True)
        return m, idx

    def body(j, carry):
        vecmax, vecidx = carry
        base = pl.multiple_of(j * CHUNK, CHUNK)
        g, col_u = chunk_vals(base, CHUNK)
        vals = g + logits_ref[:, pl.ds(base, CHUNK)]
        take = vals > vecmax
        return (jnp.maximum(vecmax, vals),
                jnp.where(take, col_u.astype(jnp.int32), vecidx))

    init = (jnp.full((BLOCK_ROWS, CHUNK), _NEG_INF, jnp.float32),
            jnp.zeros((BLOCK_ROWS, CHUNK), jnp.int32))
    vecmax, vecidx = jax.lax.fori_loop(0, NUM_FULL, body, init)
    curmax, curidx = reduce_first(vecmax, vecidx)

    g, col_u = chunk_vals(TAIL_BASE, TAIL_W)
    vals = g + logits_ref[:, pl.ds(TAIL_BASE, TAIL_W)]
    tmax, tidx = reduce_first(vals, col_u.astype(jnp.int32))
    take = tmax > curmax
    curidx = jnp.where(take, tidx, curidx)

    out_ref[0, 0, :] = curidx[:, 0]


def kernel(logits):
    out = pl.pallas_call(
        _sampler_kernel,
        grid=(ROWS // BLOCK_ROWS,),
        in_specs=[pl.BlockSpec((BLOCK_ROWS, COLS), lambda i: (i, 0))],
        out_specs=pl.BlockSpec((1, 1, BLOCK_ROWS), lambda i: (i, 0, 0)),
        out_shape=jax.ShapeDtypeStruct((ROWS // BLOCK_ROWS, 1, BLOCK_ROWS),
                                       jnp.int32),
        compiler_params=pltpu.CompilerParams(
            dimension_semantics=("arbitrary",)),
    )(logits)
    return out.reshape(ROWS)


# speedup vs baseline: 3.8996x; 1.0007x over previous
import jax
import jax.numpy as jnp
import numpy as np
from jax.experimental import pallas as pl
from jax.experimental.pallas import tpu as pltpu

ROWS = 128
COLS = 100000
BLOCK_ROWS = 8
CHUNK = 1280
NUM_FULL = COLS // CHUNK
TAIL_BASE = NUM_FULL * CHUNK
TAIL_W = COLS - TAIL_BASE

_TINY = np.float32(np.finfo(np.float32).tiny)
_NEG_INF = np.float32(-np.inf)
_INT_MAX = np.int32(2**31 - 1)


def _threefry_bits(flat_u32):
    u32 = lambda v: jnp.uint32(v)
    ks0 = u32(0)
    ks1 = u32(42)
    ks2 = ks0 ^ ks1 ^ u32(0x1BD11BDA)

    def rotl(x, d):
        return (x << u32(d)) | (x >> u32(32 - d))

    def rounds(x0, x1, rots):
        for r in rots:
            x0 = x0 + x1
            x1 = rotl(x1, r)
            x1 = x0 ^ x1
        return x0, x1

    rot_a = (13, 15, 26, 6)
    rot_b = (17, 29, 16, 24)

    x0 = jnp.zeros_like(flat_u32) + ks0
    x1 = flat_u32 + ks1
    x0, x1 = rounds(x0, x1, rot_a)
    x0 = x0 + ks1
    x1 = x1 + ks2 + u32(1)
    x0, x1 = rounds(x0, x1, rot_b)
    x0 = x0 + ks2
    x1 = x1 + ks0 + u32(2)
    x0, x1 = rounds(x0, x1, rot_a)
    x0 = x0 + ks0
    x1 = x1 + ks1 + u32(3)
    x0, x1 = rounds(x0, x1, rot_b)
    x0 = x0 + ks1
    x1 = x1 + ks2 + u32(4)
    x0, x1 = rounds(x0, x1, rot_a)
    x0 = x0 + ks2
    x1 = x1 + ks0 + u32(5)
    return x0 ^ x1


def _gumbel_from_bits(bits):
    float_bits = (bits >> jnp.uint32(9)) | jnp.uint32(0x3F800000)
    floats = jax.lax.bitcast_convert_type(float_bits, jnp.float32)
    floats = floats - jnp.float32(1.0)
    u = jnp.maximum(_TINY, floats * (jnp.float32(1.0) - _TINY) + _TINY)
    return -jnp.log(-jnp.log(u))


def _sampler_kernel(logits_ref, out_ref):
    i = pl.program_id(0)
    row0 = (i * BLOCK_ROWS).astype(jnp.uint32)
    row_iota = jax.lax.broadcasted_iota(jnp.uint32, (BLOCK_ROWS, 1), 0)
    row_base = (row0 + row_iota) * jnp.uint32(COLS)

    def chunk_vals(base, width):
        col_u = (jnp.uint32(base)
                 + jax.lax.broadcasted_iota(jnp.uint32, (BLOCK_ROWS, width), 1))
        flat = row_base + col_u
        g = _gumbel_from_bits(_threefry_bits(flat))
        return g, col_u

    def reduce_first(vecmax, vecidx):
        m = jnp.max(vecmax, axis=1, keepdims=True)
        idx = jnp.min(jnp.where(vecmax == m, vecidx, _INT_MAX),
                      axis=1, keepdims=True)
        return m, idx

    def body(j, carry):
        vecmax, vecidx = carry
        base = pl.multiple_of(j * CHUNK, CHUNK)
        g, col_u = chunk_vals(base, CHUNK)
        vals = g + logits_ref[:, pl.ds(base, CHUNK)]
        take = vals > vecmax
        return (jnp.maximum(vecmax, vals),
                jnp.where(take, col_u.astype(jnp.int32), vecidx))

    init = (jnp.full((BLOCK_ROWS, CHUNK), _NEG_INF, jnp.float32),
            jnp.zeros((BLOCK_ROWS, CHUNK), jnp.int32))
    vecmax, vecidx = jax.lax.fori_loop(0, NUM_FULL, body, init)
    curmax, curidx = reduce_first(vecmax, vecidx)

    g, col_u = chunk_vals(TAIL_BASE, TAIL_W)
    vals = g + logits_ref[:, pl.ds(TAIL_BASE, TAIL_W)]
    tmax, tidx = reduce_first(vals, col_u.astype(jnp.int32))
    take = tmax > curmax
    curidx = jnp.where(take, tidx, curidx)

    out_ref[0, 0, :] = curidx[:, 0]


def kernel(logits):
    out = pl.pallas_call(
        _sampler_kernel,
        grid=(ROWS // BLOCK_ROWS,),
        in_specs=[pl.BlockSpec((BLOCK_ROWS, COLS), lambda i: (i, 0))],
        out_specs=pl.BlockSpec((1, 1, BLOCK_ROWS), lambda i: (i, 0, 0)),
        out_shape=jax.ShapeDtypeStruct((ROWS // BLOCK_ROWS, 1, BLOCK_ROWS),
                                       jnp.int32),
        compiler_params=pltpu.CompilerParams(
            dimension_semantics=("parallel",)),
    )(logits)
    return out.reshape(ROWS)
